# Initial kernel scaffold; baseline (speedup 1.0000x reference)
import jax
import jax.numpy as jnp
from jax.experimental import pallas as pl


def kernel(pred_logits, pred_boxes, target_sizes):
    raise NotImplementedError("write your pallas kernel here")


# speedup vs baseline: 5.5626x; 5.5626x over previous
import functools

import jax
import jax.numpy as jnp
from jax import lax
from jax.experimental import pallas as pl
from jax.experimental.pallas import tpu as pltpu
from jax.experimental.pallas import tpu_sc as plsc

B = 64
Q = 900
C = 91
N = Q * C
NPAD = 81904
NV = NPAD // 16
K = 300
OUTP = 304

NB1 = 512
NB2 = 1024
CAP1 = 4096
CAP2 = 512
MINKEY = -2**31

NC = 2
NS = 16


def _body(prob_hbm, boxes_hbm, ts_hbm, oscore_hbm, olab_hbm, obox_hbm,
          sc_raw, hist1, hist2, tot1, ctot1, tot2, ctot2,
          ckey, cidx, dkey, didx, vm, out_key, out_idx,
          slab, sbox, boxes_v, ts_v):
  wid = lax.axis_index("s") * NC + lax.axis_index("c")
  iota = lax.iota(jnp.int32, 16)
  ones = jnp.ones((16,), jnp.int32)
  zeros_i = jnp.zeros((16,), jnp.int32)
  lane1 = iota * NB1
  lane2 = iota * NB2

  def scan_desc(tot_ref, ctot_ref, nb, kk):
    nch = nb // 16
    nchv = nch // 16

    def chunk_body(jj, carry):
      acc, cmax, amax = carry
      jc = nchv - 1 - jj
      t = ctot_ref[pl.ds(16 * jc, 16)]
      td = jnp.flip(t)
      cs = plsc.cumsum(td)
      ae = cs - td
      ag = acc + ae
      hit = (ag < kk) & (ag + td >= kk)
      ids = 16 * jc + 15 - iota
      cmax = jnp.maximum(cmax, jnp.max(jnp.where(hit, ids, -1)))
      amax = jnp.maximum(amax, jnp.max(jnp.where(hit, ag, -1)))
      return acc + jnp.sum(t), cmax, amax

    _, cstar, astar = lax.fori_loop(
        0, nchv, chunk_body,
        (jnp.int32(0), jnp.int32(-1), jnp.int32(-1)))
    t = tot_ref[pl.ds(16 * cstar, 16)]
    td = jnp.flip(t)
    cs = plsc.cumsum(td)
    ae = cs - td
    ag = astar + ae
    hit = (ag < kk) & (ag + td >= kk)
    buck = 16 * cstar + 15 - iota
    bstar = jnp.max(jnp.where(hit, buck, -1))
    above = jnp.max(jnp.where(hit, ag, -1))
    return bstar, above

  def sum_hist(hist_ref, tot_ref, ctot_ref, nb):
    nch = nb // 16

    def body(j, _):
      acc = hist_ref[pl.ds(16 * j, 16)]
      for l in range(1, 16):
        acc = acc + hist_ref[pl.ds(l * nb + 16 * j, 16)]
      tot_ref[pl.ds(16 * j, 16)] = acc
      jq = (j >> 4) * 16
      cv = ctot_ref[pl.ds(jq, 16)]
      ctot_ref[pl.ds(jq, 16)] = jnp.where(iota == (j & 15), jnp.sum(acc), cv)
      return 0

    lax.fori_loop(0, nch, body, 0)

  def process(img):
    pltpu.sync_copy(prob_hbm.at[img], sc_raw)
    pltpu.sync_copy(boxes_hbm.at[img], boxes_v)
    pltpu.sync_copy(ts_hbm.at[img], ts_v)

    def zero1(j, _):
      hist1[pl.ds(16 * j, 16)] = zeros_i
      return 0
    lax.fori_loop(0, (16 * NB1) // 16, zero1, 0)

    def zero2(j, _):
      hist2[pl.ds(16 * j, 16)] = zeros_i
      return 0
    lax.fori_loop(0, (16 * NB2) // 16, zero2, 0)

    mink = jnp.full((16,), MINKEY, jnp.int32)

    def pre1(j, _):
      ckey[pl.ds(16 * j, 16)] = mink
      cidx[pl.ds(16 * j, 16)] = zeros_i
      return 0
    lax.fori_loop(0, (CAP1 + 16) // 16, pre1, 0)

    def pre2(j, _):
      dkey[pl.ds(16 * j, 16)] = mink
      didx[pl.ds(16 * j, 16)] = zeros_i
      return 0
    lax.fori_loop(0, (CAP2 + 16) // 16, pre2, 0)

    for j in range(OUTP // 16):
      out_key[pl.ds(16 * j, 16)] = zeros_i
      out_idx[pl.ds(16 * j, 16)] = zeros_i

    def l1_body(i, _):
      k = sc_raw[pl.ds(16 * i, 16)]
      bk = jnp.maximum(k >> 21, 0)
      plsc.addupdate_scatter(hist1, [lane1 + bk], ones)
      return 0
    lax.fori_loop(0, NV, l1_body, 0)

    sum_hist(hist1, tot1, ctot1, NB1)
    b1, a1 = scan_desc(tot1, ctot1, NB1, jnp.int32(K))

    capv1 = jnp.full((16,), CAP1, jnp.int32)

    def c1_body(i, wv):
      k = sc_raw[pl.ds(16 * i, 16)]
      m = ((k >> 21) >= b1) & (wv < capv1)
      mi = m.astype(jnp.int32)
      pos = wv + plsc.cumsum(mi) - mi
      plsc.store_scatter(ckey, [pos], k, mask=m)
      plsc.store_scatter(cidx, [pos], 16 * i + iota, mask=m)
      return wv + plsc.all_reduce_population_count(m)
    lax.fori_loop(0, NV, c1_body, zeros_i)

    def l2_body(i, _):
      kv = ckey[pl.ds(16 * i, 16)]
      inb = (kv >> 21) == b1
      b2k = (kv >> 11) & (NB2 - 1)
      plsc.addupdate_scatter(hist2, [lane2 + b2k], ones, mask=inb)
      return 0
    lax.fori_loop(0, CAP1 // 16, l2_body, 0)

    sum_hist(hist2, tot2, ctot2, NB2)
    b2, a2 = scan_desc(tot2, ctot2, NB2, K - a1)

    p21 = b1 * NB2 + b2
    capv2 = jnp.full((16,), CAP2, jnp.int32)

    def c2_body(i, wv):
      kv = ckey[pl.ds(16 * i, 16)]
      m = ((kv >> 11) >= p21) & (wv < capv2)
      mi = m.astype(jnp.int32)
      pos = wv + plsc.cumsum(mi) - mi
      plsc.store_scatter(dkey, [pos], kv, mask=m)
      iv = cidx[pl.ds(16 * i, 16)]
      plsc.store_scatter(didx, [pos], iv, mask=m)
      return wv + plsc.all_reduce_population_count(m)
    lax.fori_loop(0, CAP1 // 16, c2_body, zeros_i)

    NVD = (CAP2 + 16) // 16
    NVM = 48

    for j in range(NVM // 16):
      vm[pl.ds(16 * j, 16)] = mink

    def vm_body(j, _):
      mx = jnp.max(dkey[pl.ds(16 * j, 16)])
      jq = (j >> 4) * 16
      vv = vm[pl.ds(jq, 16)]
      vm[pl.ds(jq, 16)] = jnp.where(iota == (j & 15), mx, vv)
      return 0
    lax.fori_loop(0, NVD, vm_body, 0)

    big = jnp.full((16,), 999, jnp.int32)

    def sel_body(t, _):
      macc = vm[pl.ds(0, 16)]
      for jv in range(1, NVM // 16):
        macc = jnp.maximum(macc, vm[pl.ds(16 * jv, 16)])
      m = jnp.max(macc)
      pacc = big
      for jv in range(NVM // 16):
        hv = vm[pl.ds(16 * jv, 16)]
        pacc = jnp.minimum(pacc, jnp.where(hv == m, iota + 16 * jv, big))
      jsel = jnp.min(pacc)
      v = dkey[pl.ds(16 * jsel, 16)]
      lp = jnp.min(jnp.where(v == m, iota, big))
      pos = 16 * jsel + lp
      v2 = jnp.where(iota == lp, MINKEY, v)
      dkey[pl.ds(16 * jsel, 16)] = v2
      newm = jnp.max(v2)
      for jv in range(NVM // 16):
        vv = vm[pl.ds(16 * jv, 16)]
        vm[pl.ds(16 * jv, 16)] = jnp.where((iota + 16 * jv) == jsel, newm, vv)
      tq = (t >> 4) * 16
      tl = t & 15
      okv = out_key[pl.ds(tq, 16)]
      out_key[pl.ds(tq, 16)] = jnp.where(iota == tl, m, okv)
      oiv = out_idx[pl.ds(tq, 16)]
      selidx = plsc.load_gather(didx, [jnp.full((16,), pos, jnp.int32)])
      out_idx[pl.ds(tq, 16)] = jnp.where(iota == tl, selidx, oiv)
      return 0
    lax.fori_loop(0, K, sel_body, 0)

    tv = ts_v[pl.ds(0, 16)]
    hf = tv[0].astype(jnp.float32)
    wf = tv[1].astype(jnp.float32)
    wvv = jnp.full((16,), wf)
    hvv = jnp.full((16,), hf)
    c0 = zeros_i
    c1c = ones
    c2c = jnp.full((16,), 2, jnp.int32)
    c3c = jnp.full((16,), 3, jnp.int32)

    for o in range(OUTP // 16):
      idx = out_idx[pl.ds(16 * o, 16)]
      q0 = (idx * 5762) >> 19
      r = idx - q0 * 91
      qq = q0 + (r >> 31)
      slab[pl.ds(16 * o, 16)] = idx - qq * 91
      cx = plsc.load_gather(boxes_v, [qq, c0])
      cy = plsc.load_gather(boxes_v, [qq, c1c])
      bw = plsc.load_gather(boxes_v, [qq, c2c])
      bh = plsc.load_gather(boxes_v, [qq, c3c])
      x0 = (cx - 0.5 * bw) * wvv
      y0 = (cy - 0.5 * bh) * hvv
      x1 = (cx + 0.5 * bw) * wvv
      y1 = (cy + 0.5 * bh) * hvv
      base4 = 64 * o + iota * 4
      plsc.store_scatter(sbox, [base4], x0)
      plsc.store_scatter(sbox, [base4 + 1], y0)
      plsc.store_scatter(sbox, [base4 + 2], x1)
      plsc.store_scatter(sbox, [base4 + 3], y1)

    pltpu.sync_copy(out_key, oscore_hbm.at[img])
    pltpu.sync_copy(slab, olab_hbm.at[img])
    pltpu.sync_copy(sbox, obox_hbm.at[img])

  def img_body(t, _):
    process(wid * 2 + t)
    return 0
  lax.fori_loop(0, 2, img_body, 0)


@functools.partial(
    pl.kernel,
    out_type=(
        jax.ShapeDtypeStruct((B, OUTP), jnp.int32),
        jax.ShapeDtypeStruct((B, OUTP), jnp.int32),
        jax.ShapeDtypeStruct((B, 4 * OUTP), jnp.float32),
    ),
    mesh=plsc.VectorSubcoreMesh(core_axis_name="c", subcore_axis_name="s"),
    compiler_params=pltpu.CompilerParams(
        needs_layout_passes=False, use_tc_tiling_on_sc=False),
    scratch_types=[
        pltpu.VMEM((NPAD,), jnp.int32),
        pltpu.VMEM((16 * NB1,), jnp.int32),
        pltpu.VMEM((16 * NB2,), jnp.int32),
        pltpu.VMEM((NB1,), jnp.int32),
        pltpu.VMEM((NB1 // 16,), jnp.int32),
        pltpu.VMEM((NB2,), jnp.int32),
        pltpu.VMEM((NB2 // 16,), jnp.int32),
        pltpu.VMEM((CAP1 + 16,), jnp.int32),
        pltpu.VMEM((CAP1 + 16,), jnp.int32),
        pltpu.VMEM((CAP2 + 16,), jnp.int32),
        pltpu.VMEM((CAP2 + 32,), jnp.int32),
        pltpu.VMEM((48,), jnp.int32),
        pltpu.VMEM((OUTP,), jnp.int32),
        pltpu.VMEM((OUTP,), jnp.int32),
        pltpu.VMEM((OUTP,), jnp.int32),
        pltpu.VMEM((4 * OUTP,), jnp.float32),
        pltpu.VMEM((Q, 4), jnp.float32),
        pltpu.VMEM((16,), jnp.int32),
    ],
)
def _postprocess_sc(prob_hbm, boxes_hbm, ts_hbm,
                    oscore_hbm, olab_hbm, obox_hbm, *scratch):
  _body(prob_hbm, boxes_hbm, ts_hbm, oscore_hbm, olab_hbm, obox_hbm, *scratch)


def kernel(pred_logits, pred_boxes, target_sizes):
  prob = jax.nn.sigmoid(pred_logits).reshape(B, N)
  prob = jnp.pad(prob, ((0, 0), (0, NPAD - N)), constant_values=-1.0)
  pbits = lax.bitcast_convert_type(prob, jnp.int32)
  ts = jnp.pad(target_sizes, ((0, 0), (0, 14)))
  skey, labels, boxes = _postprocess_sc(pbits, pred_boxes, ts)
  scores = lax.bitcast_convert_type(skey[:, :K], jnp.float32)
  return (scores, labels[:, :K],
          boxes.reshape(B, OUTP, 4)[:, :K, :])


# speedup vs baseline: 7.7776x; 1.3982x over previous
import functools

import jax
import jax.numpy as jnp
from jax import lax
from jax.experimental import pallas as pl
from jax.experimental.pallas import tpu as pltpu
from jax.experimental.pallas import tpu_sc as plsc

B = 64
Q = 900
C = 91
N = Q * C
NPAD = 81904
NV = NPAD // 16
K = 300
OUTP = 304

NB1 = 512
NB2 = 1024
CAP1 = 4096
CAP2 = 512
MINKEY = -2**31

NC = 2
NS = 16


def _body(prob_hbm, boxes_hbm, ts_hbm, oscore_hbm, olab_hbm, obox_hbm,
          sc_raw, hist1, hist2, tot1, ctot1, tot2, ctot2,
          ckey, cidx, dkey, didx, vm, out_key, out_idx,
          slab, sbox, boxes_v, ts_v):
  wid = lax.axis_index("s") * NC + lax.axis_index("c")
  iota = lax.iota(jnp.int32, 16)
  ones = jnp.ones((16,), jnp.int32)
  zeros_i = jnp.zeros((16,), jnp.int32)
  lane1 = iota * NB1
  lane2 = iota * NB2

  def scan_desc(tot_ref, ctot_ref, nb, kk):
    nch = nb // 16
    nchv = nch // 16

    def chunk_body(jj, carry):
      acc, cmax, amax = carry
      jc = nchv - 1 - jj
      t = ctot_ref[pl.ds(16 * jc, 16)]
      td = jnp.flip(t)
      cs = plsc.cumsum(td)
      ae = cs - td
      ag = acc + ae
      hit = (ag < kk) & (ag + td >= kk)
      ids = 16 * jc + 15 - iota
      cmax = jnp.maximum(cmax, jnp.max(jnp.where(hit, ids, -1)))
      amax = jnp.maximum(amax, jnp.max(jnp.where(hit, ag, -1)))
      return acc + jnp.sum(t), cmax, amax

    _, cstar, astar = lax.fori_loop(
        0, nchv, chunk_body,
        (jnp.int32(0), jnp.int32(-1), jnp.int32(-1)))
    t = tot_ref[pl.ds(16 * cstar, 16)]
    td = jnp.flip(t)
    cs = plsc.cumsum(td)
    ae = cs - td
    ag = astar + ae
    hit = (ag < kk) & (ag + td >= kk)
    buck = 16 * cstar + 15 - iota
    bstar = jnp.max(jnp.where(hit, buck, -1))
    above = jnp.max(jnp.where(hit, ag, -1))
    return bstar, above

  def sum_hist(hist_ref, tot_ref, ctot_ref, nb):
    nch = nb // 16

    def body(j, _):
      acc = hist_ref[pl.ds(16 * j, 16)]
      for l in range(1, 16):
        acc = acc + hist_ref[pl.ds(l * nb + 16 * j, 16)]
      tot_ref[pl.ds(16 * j, 16)] = acc
      jq = (j >> 4) * 16
      cv = ctot_ref[pl.ds(jq, 16)]
      ctot_ref[pl.ds(jq, 16)] = jnp.where(iota == (j & 15), jnp.sum(acc), cv)
      return 0

    lax.fori_loop(0, nch, body, 0)

  def process(img):
    pltpu.sync_copy(prob_hbm.at[img], sc_raw)
    pltpu.sync_copy(boxes_hbm.at[img], boxes_v)
    pltpu.sync_copy(ts_hbm.at[img], ts_v)

    @plsc.parallel_loop(0, (16 * NB1) // 16, unroll=8)
    def _(j):
      hist1[pl.ds(16 * j, 16)] = zeros_i

    @plsc.parallel_loop(0, (16 * NB2) // 16, unroll=8)
    def _(j):
      hist2[pl.ds(16 * j, 16)] = zeros_i

    mink = jnp.full((16,), MINKEY, jnp.int32)

    @plsc.parallel_loop(0, (CAP1 + 16) // 16, unroll=8)
    def _(j):
      ckey[pl.ds(16 * j, 16)] = mink
      cidx[pl.ds(16 * j, 16)] = zeros_i

    @plsc.parallel_loop(0, (CAP2 + 16) // 16, unroll=8)
    def _(j):
      dkey[pl.ds(16 * j, 16)] = mink
      didx[pl.ds(16 * j, 16)] = zeros_i

    for j in range(OUTP // 16):
      out_key[pl.ds(16 * j, 16)] = zeros_i
      out_idx[pl.ds(16 * j, 16)] = zeros_i

    @plsc.parallel_loop(0, NV, unroll=4)
    def _(i):
      k = sc_raw[pl.ds(16 * i, 16)]
      bk = jnp.maximum(k >> 21, 0)
      plsc.addupdate_scatter(hist1, [lane1 + bk], ones)

    sum_hist(hist1, tot1, ctot1, NB1)
    b1, a1 = scan_desc(tot1, ctot1, NB1, jnp.int32(K))

    capv1 = jnp.full((16,), CAP1, jnp.int32)

    @plsc.parallel_loop(0, NV, unroll=4, carry=zeros_i)
    def _(i, wv):
      k = sc_raw[pl.ds(16 * i, 16)]
      m = ((k >> 21) >= b1) & (wv < capv1)
      mi = m.astype(jnp.int32)
      pos = wv + plsc.cumsum(mi) - mi
      plsc.store_scatter(ckey, [pos], k, mask=m)
      plsc.store_scatter(cidx, [pos], 16 * i + iota, mask=m)
      return wv + plsc.all_reduce_population_count(m)

    @plsc.parallel_loop(0, CAP1 // 16, unroll=4)
    def _(i):
      kv = ckey[pl.ds(16 * i, 16)]
      inb = (kv >> 21) == b1
      b2k = (kv >> 11) & (NB2 - 1)
      plsc.addupdate_scatter(hist2, [lane2 + b2k], ones, mask=inb)

    sum_hist(hist2, tot2, ctot2, NB2)
    b2, a2 = scan_desc(tot2, ctot2, NB2, K - a1)

    p21 = b1 * NB2 + b2
    capv2 = jnp.full((16,), CAP2, jnp.int32)

    @plsc.parallel_loop(0, CAP1 // 16, unroll=4, carry=zeros_i)
    def _(i, wv):
      kv = ckey[pl.ds(16 * i, 16)]
      m = ((kv >> 11) >= p21) & (wv < capv2)
      mi = m.astype(jnp.int32)
      pos = wv + plsc.cumsum(mi) - mi
      plsc.store_scatter(dkey, [pos], kv, mask=m)
      iv = cidx[pl.ds(16 * i, 16)]
      plsc.store_scatter(didx, [pos], iv, mask=m)
      return wv + plsc.all_reduce_population_count(m)

    NVD = (CAP2 + 16) // 16
    NVM = 48

    for j in range(NVM // 16):
      vm[pl.ds(16 * j, 16)] = mink

    def vm_body(j, _):
      mx = jnp.max(dkey[pl.ds(16 * j, 16)])
      jq = (j >> 4) * 16
      vv = vm[pl.ds(jq, 16)]
      vm[pl.ds(jq, 16)] = jnp.where(iota == (j & 15), mx, vv)
      return 0
    lax.fori_loop(0, NVD, vm_body, 0)

    big = jnp.full((16,), 999, jnp.int32)

    def sel_body(t, _):
      macc = vm[pl.ds(0, 16)]
      for jv in range(1, NVM // 16):
        macc = jnp.maximum(macc, vm[pl.ds(16 * jv, 16)])
      m = jnp.max(macc)
      pacc = big
      for jv in range(NVM // 16):
        hv = vm[pl.ds(16 * jv, 16)]
        pacc = jnp.minimum(pacc, jnp.where(hv == m, iota + 16 * jv, big))
      jsel = jnp.min(pacc)
      v = dkey[pl.ds(16 * jsel, 16)]
      lp = jnp.min(jnp.where(v == m, iota, big))
      pos = 16 * jsel + lp
      v2 = jnp.where(iota == lp, MINKEY, v)
      dkey[pl.ds(16 * jsel, 16)] = v2
      newm = jnp.max(v2)
      for jv in range(NVM // 16):
        vv = vm[pl.ds(16 * jv, 16)]
        vm[pl.ds(16 * jv, 16)] = jnp.where((iota + 16 * jv) == jsel, newm, vv)
      tq = (t >> 4) * 16
      tl = t & 15
      okv = out_key[pl.ds(tq, 16)]
      out_key[pl.ds(tq, 16)] = jnp.where(iota == tl, m, okv)
      oiv = out_idx[pl.ds(tq, 16)]
      selidx = plsc.load_gather(didx, [jnp.full((16,), pos, jnp.int32)])
      out_idx[pl.ds(tq, 16)] = jnp.where(iota == tl, selidx, oiv)
      return 0
    lax.fori_loop(0, K, sel_body, 0)

    tv = ts_v[pl.ds(0, 16)]
    hf = tv[0].astype(jnp.float32)
    wf = tv[1].astype(jnp.float32)
    wvv = jnp.full((16,), wf)
    hvv = jnp.full((16,), hf)
    c0 = zeros_i
    c1c = ones
    c2c = jnp.full((16,), 2, jnp.int32)
    c3c = jnp.full((16,), 3, jnp.int32)

    for o in range(OUTP // 16):
      idx = out_idx[pl.ds(16 * o, 16)]
      q0 = (idx * 5762) >> 19
      r = idx - q0 * 91
      qq = q0 + (r >> 31)
      slab[pl.ds(16 * o, 16)] = idx - qq * 91
      cx = plsc.load_gather(boxes_v, [qq, c0])
      cy = plsc.load_gather(boxes_v, [qq, c1c])
      bw = plsc.load_gather(boxes_v, [qq, c2c])
      bh = plsc.load_gather(boxes_v, [qq, c3c])
      x0 = (cx - 0.5 * bw) * wvv
      y0 = (cy - 0.5 * bh) * hvv
      x1 = (cx + 0.5 * bw) * wvv
      y1 = (cy + 0.5 * bh) * hvv
      base4 = 64 * o + iota * 4
      plsc.store_scatter(sbox, [base4], x0)
      plsc.store_scatter(sbox, [base4 + 1], y0)
      plsc.store_scatter(sbox, [base4 + 2], x1)
      plsc.store_scatter(sbox, [base4 + 3], y1)

    pltpu.sync_copy(out_key, oscore_hbm.at[img])
    pltpu.sync_copy(slab, olab_hbm.at[img])
    pltpu.sync_copy(sbox, obox_hbm.at[img])

  def img_body(t, _):
    process(wid * 2 + t)
    return 0
  lax.fori_loop(0, 2, img_body, 0)


@functools.partial(
    pl.kernel,
    out_type=(
        jax.ShapeDtypeStruct((B, OUTP), jnp.int32),
        jax.ShapeDtypeStruct((B, OUTP), jnp.int32),
        jax.ShapeDtypeStruct((B, 4 * OUTP), jnp.float32),
    ),
    mesh=plsc.VectorSubcoreMesh(core_axis_name="c", subcore_axis_name="s"),
    compiler_params=pltpu.CompilerParams(
        needs_layout_passes=False, use_tc_tiling_on_sc=False),
    scratch_types=[
        pltpu.VMEM((NPAD,), jnp.int32),
        pltpu.VMEM((16 * NB1,), jnp.int32),
        pltpu.VMEM((16 * NB2,), jnp.int32),
        pltpu.VMEM((NB1,), jnp.int32),
        pltpu.VMEM((NB1 // 16,), jnp.int32),
        pltpu.VMEM((NB2,), jnp.int32),
        pltpu.VMEM((NB2 // 16,), jnp.int32),
        pltpu.VMEM((CAP1 + 16,), jnp.int32),
        pltpu.VMEM((CAP1 + 16,), jnp.int32),
        pltpu.VMEM((CAP2 + 16,), jnp.int32),
        pltpu.VMEM((CAP2 + 32,), jnp.int32),
        pltpu.VMEM((48,), jnp.int32),
        pltpu.VMEM((OUTP,), jnp.int32),
        pltpu.VMEM((OUTP,), jnp.int32),
        pltpu.VMEM((OUTP,), jnp.int32),
        pltpu.VMEM((4 * OUTP,), jnp.float32),
        pltpu.VMEM((Q, 4), jnp.float32),
        pltpu.VMEM((16,), jnp.int32),
    ],
)
def _postprocess_sc(prob_hbm, boxes_hbm, ts_hbm,
                    oscore_hbm, olab_hbm, obox_hbm, *scratch):
  _body(prob_hbm, boxes_hbm, ts_hbm, oscore_hbm, olab_hbm, obox_hbm, *scratch)


def kernel(pred_logits, pred_boxes, target_sizes):
  prob = jax.nn.sigmoid(pred_logits).reshape(B, N)
  prob = jnp.pad(prob, ((0, 0), (0, NPAD - N)), constant_values=-1.0)
  pbits = lax.bitcast_convert_type(prob, jnp.int32)
  ts = jnp.pad(target_sizes, ((0, 0), (0, 14)))
  skey, labels, boxes = _postprocess_sc(pbits, pred_boxes, ts)
  scores = lax.bitcast_convert_type(skey[:, :K], jnp.float32)
  return (scores, labels[:, :K],
          boxes.reshape(B, OUTP, 4)[:, :K, :])


# speedup vs baseline: 7.7957x; 1.0023x over previous
import functools

import jax
import jax.numpy as jnp
from jax import lax
from jax.experimental import pallas as pl
from jax.experimental.pallas import tpu as pltpu
from jax.experimental.pallas import tpu_sc as plsc

B = 64
Q = 900
C = 91
N = Q * C
NPAD = 81904
NV = NPAD // 16
K = 300
OUTP = 304

NB1 = 512
NB2 = 1024
CAP1 = 4096
CAP2 = 512
MINKEY = -2**31

NC = 2
NS = 16


def _body(prob_hbm, boxes_hbm, ts_hbm, oscore_hbm, olab_hbm, obox_hbm,
          sc_raw, hist1, hist2, tot1, ctot1, tot2, ctot2,
          ckey, cidx, dkey, didx, vm, out_key, out_idx,
          slab, sbox, boxes_v, ts_v):
  wid = lax.axis_index("s") * NC + lax.axis_index("c")
  iota = lax.iota(jnp.int32, 16)
  ones = jnp.ones((16,), jnp.int32)
  zeros_i = jnp.zeros((16,), jnp.int32)
  lane1 = iota * NB1
  lane2 = iota * NB2

  def scan_desc(tot_ref, ctot_ref, nb, kk):
    nch = nb // 16
    nchv = nch // 16

    def chunk_body(jj, carry):
      acc, cmax, amax = carry
      jc = nchv - 1 - jj
      t = ctot_ref[pl.ds(16 * jc, 16)]
      td = jnp.flip(t)
      cs = plsc.cumsum(td)
      ae = cs - td
      ag = acc + ae
      hit = (ag < kk) & (ag + td >= kk)
      ids = 16 * jc + 15 - iota
      cmax = jnp.maximum(cmax, jnp.max(jnp.where(hit, ids, -1)))
      amax = jnp.maximum(amax, jnp.max(jnp.where(hit, ag, -1)))
      return acc + jnp.sum(t), cmax, amax

    _, cstar, astar = lax.fori_loop(
        0, nchv, chunk_body,
        (jnp.int32(0), jnp.int32(-1), jnp.int32(-1)))
    t = tot_ref[pl.ds(16 * cstar, 16)]
    td = jnp.flip(t)
    cs = plsc.cumsum(td)
    ae = cs - td
    ag = astar + ae
    hit = (ag < kk) & (ag + td >= kk)
    buck = 16 * cstar + 15 - iota
    bstar = jnp.max(jnp.where(hit, buck, -1))
    above = jnp.max(jnp.where(hit, ag, -1))
    return bstar, above

  def sum_hist(hist_ref, tot_ref, ctot_ref, nb):
    nch = nb // 16

    def body(j, _):
      acc = hist_ref[pl.ds(16 * j, 16)]
      for l in range(1, 16):
        acc = acc + hist_ref[pl.ds(l * nb + 16 * j, 16)]
      tot_ref[pl.ds(16 * j, 16)] = acc
      jq = (j >> 4) * 16
      cv = ctot_ref[pl.ds(jq, 16)]
      ctot_ref[pl.ds(jq, 16)] = jnp.where(iota == (j & 15), jnp.sum(acc), cv)
      return 0

    lax.fori_loop(0, nch, body, 0)

  def process(img):
    pltpu.sync_copy(prob_hbm.at[img], sc_raw)
    pltpu.sync_copy(boxes_hbm.at[img], boxes_v)
    pltpu.sync_copy(ts_hbm.at[img], ts_v)

    @plsc.parallel_loop(0, (16 * NB1) // 16, unroll=8)
    def _(j):
      hist1[pl.ds(16 * j, 16)] = zeros_i

    @plsc.parallel_loop(0, (16 * NB2) // 16, unroll=8)
    def _(j):
      hist2[pl.ds(16 * j, 16)] = zeros_i

    mink = jnp.full((16,), MINKEY, jnp.int32)

    @plsc.parallel_loop(0, (CAP1 + 16) // 16, unroll=8)
    def _(j):
      ckey[pl.ds(16 * j, 16)] = mink
      cidx[pl.ds(16 * j, 16)] = zeros_i

    @plsc.parallel_loop(0, (CAP2 + 16) // 16, unroll=8)
    def _(j):
      dkey[pl.ds(16 * j, 16)] = mink
      didx[pl.ds(16 * j, 16)] = zeros_i

    for j in range(OUTP // 16):
      out_key[pl.ds(16 * j, 16)] = zeros_i
      out_idx[pl.ds(16 * j, 16)] = zeros_i

    @plsc.parallel_loop(0, NV, unroll=8)
    def _(i):
      k = sc_raw[pl.ds(16 * i, 16)]
      bk = jnp.maximum(k >> 21, 0)
      plsc.addupdate_scatter(hist1, [lane1 + bk], ones)

    sum_hist(hist1, tot1, ctot1, NB1)
    b1, a1 = scan_desc(tot1, ctot1, NB1, jnp.int32(K))

    capv1 = jnp.full((16,), CAP1, jnp.int32)

    @plsc.parallel_loop(0, NV, unroll=8, carry=zeros_i)
    def _(i, wv):
      k = sc_raw[pl.ds(16 * i, 16)]
      m = ((k >> 21) >= b1) & (wv < capv1)
      mi = m.astype(jnp.int32)
      pos = wv + plsc.cumsum(mi) - mi
      plsc.store_scatter(ckey, [pos], k, mask=m)
      plsc.store_scatter(cidx, [pos], 16 * i + iota, mask=m)
      return wv + plsc.all_reduce_population_count(m)

    @plsc.parallel_loop(0, CAP1 // 16, unroll=4)
    def _(i):
      kv = ckey[pl.ds(16 * i, 16)]
      inb = (kv >> 21) == b1
      b2k = (kv >> 11) & (NB2 - 1)
      plsc.addupdate_scatter(hist2, [lane2 + b2k], ones, mask=inb)

    sum_hist(hist2, tot2, ctot2, NB2)
    b2, a2 = scan_desc(tot2, ctot2, NB2, K - a1)

    p21 = b1 * NB2 + b2
    capv2 = jnp.full((16,), CAP2, jnp.int32)

    @plsc.parallel_loop(0, CAP1 // 16, unroll=4, carry=zeros_i)
    def _(i, wv):
      kv = ckey[pl.ds(16 * i, 16)]
      m = ((kv >> 11) >= p21) & (wv < capv2)
      mi = m.astype(jnp.int32)
      pos = wv + plsc.cumsum(mi) - mi
      plsc.store_scatter(dkey, [pos], kv, mask=m)
      iv = cidx[pl.ds(16 * i, 16)]
      plsc.store_scatter(didx, [pos], iv, mask=m)
      return wv + plsc.all_reduce_population_count(m)

    NVD = (CAP2 + 16) // 16
    NVM = 32

    for j in range(NVM // 16):
      vm[pl.ds(16 * j, 16)] = mink

    def vm_body(j, _):
      mx = jnp.max(dkey[pl.ds(16 * j, 16)])
      jq = (j >> 4) * 16
      vv = vm[pl.ds(jq, 16)]
      vm[pl.ds(jq, 16)] = jnp.where(iota == (j & 15), mx, vv)
      return 0
    lax.fori_loop(0, NVD, vm_body, 0)

    big = jnp.full((16,), 999, jnp.int32)

    def sel_body(t, _):
      macc = vm[pl.ds(0, 16)]
      for jv in range(1, NVM // 16):
        macc = jnp.maximum(macc, vm[pl.ds(16 * jv, 16)])
      m = jnp.max(macc)
      pacc = big
      for jv in range(NVM // 16):
        hv = vm[pl.ds(16 * jv, 16)]
        pacc = jnp.minimum(pacc, jnp.where(hv == m, iota + 16 * jv, big))
      jsel = jnp.min(pacc)
      v = dkey[pl.ds(16 * jsel, 16)]
      lp = jnp.min(jnp.where(v == m, iota, big))
      pos = 16 * jsel + lp
      v2 = jnp.where(iota == lp, MINKEY, v)
      dkey[pl.ds(16 * jsel, 16)] = v2
      newm = jnp.max(v2)
      for jv in range(NVM // 16):
        vv = vm[pl.ds(16 * jv, 16)]
        vm[pl.ds(16 * jv, 16)] = jnp.where((iota + 16 * jv) == jsel, newm, vv)
      tq = (t >> 4) * 16
      tl = t & 15
      okv = out_key[pl.ds(tq, 16)]
      out_key[pl.ds(tq, 16)] = jnp.where(iota == tl, m, okv)
      oiv = out_idx[pl.ds(tq, 16)]
      selidx = plsc.load_gather(didx, [jnp.full((16,), pos, jnp.int32)])
      out_idx[pl.ds(tq, 16)] = jnp.where(iota == tl, selidx, oiv)
      return 0
    lax.fori_loop(0, K, sel_body, 0)

    tv = ts_v[pl.ds(0, 16)]
    hf = tv[0].astype(jnp.float32)
    wf = tv[1].astype(jnp.float32)
    wvv = jnp.full((16,), wf)
    hvv = jnp.full((16,), hf)
    c0 = zeros_i
    c1c = ones
    c2c = jnp.full((16,), 2, jnp.int32)
    c3c = jnp.full((16,), 3, jnp.int32)

    for o in range(OUTP // 16):
      idx = out_idx[pl.ds(16 * o, 16)]
      q0 = (idx * 5762) >> 19
      r = idx - q0 * 91
      qq = q0 + (r >> 31)
      slab[pl.ds(16 * o, 16)] = idx - qq * 91
      cx = plsc.load_gather(boxes_v, [qq, c0])
      cy = plsc.load_gather(boxes_v, [qq, c1c])
      bw = plsc.load_gather(boxes_v, [qq, c2c])
      bh = plsc.load_gather(boxes_v, [qq, c3c])
      x0 = (cx - 0.5 * bw) * wvv
      y0 = (cy - 0.5 * bh) * hvv
      x1 = (cx + 0.5 * bw) * wvv
      y1 = (cy + 0.5 * bh) * hvv
      base4 = 64 * o + iota * 4
      plsc.store_scatter(sbox, [base4], x0)
      plsc.store_scatter(sbox, [base4 + 1], y0)
      plsc.store_scatter(sbox, [base4 + 2], x1)
      plsc.store_scatter(sbox, [base4 + 3], y1)

    pltpu.sync_copy(out_key, oscore_hbm.at[img])
    pltpu.sync_copy(slab, olab_hbm.at[img])
    pltpu.sync_copy(sbox, obox_hbm.at[img])

  def img_body(t, _):
    process(wid * 2 + t)
    return 0
  lax.fori_loop(0, 2, img_body, 0)


@functools.partial(
    pl.kernel,
    out_type=(
        jax.ShapeDtypeStruct((B, OUTP), jnp.int32),
        jax.ShapeDtypeStruct((B, OUTP), jnp.int32),
        jax.ShapeDtypeStruct((B, 4 * OUTP), jnp.float32),
    ),
    mesh=plsc.VectorSubcoreMesh(core_axis_name="c", subcore_axis_name="s"),
    compiler_params=pltpu.CompilerParams(
        needs_layout_passes=False, use_tc_tiling_on_sc=False),
    scratch_types=[
        pltpu.VMEM((NPAD,), jnp.int32),
        pltpu.VMEM((16 * NB1,), jnp.int32),
        pltpu.VMEM((16 * NB2,), jnp.int32),
        pltpu.VMEM((NB1,), jnp.int32),
        pltpu.VMEM((NB1 // 16,), jnp.int32),
        pltpu.VMEM((NB2,), jnp.int32),
        pltpu.VMEM((NB2 // 16,), jnp.int32),
        pltpu.VMEM((CAP1 + 16,), jnp.int32),
        pltpu.VMEM((CAP1 + 16,), jnp.int32),
        pltpu.VMEM((CAP2 + 16,), jnp.int32),
        pltpu.VMEM((CAP2 + 32,), jnp.int32),
        pltpu.VMEM((32,), jnp.int32),
        pltpu.VMEM((OUTP,), jnp.int32),
        pltpu.VMEM((OUTP,), jnp.int32),
        pltpu.VMEM((OUTP,), jnp.int32),
        pltpu.VMEM((4 * OUTP,), jnp.float32),
        pltpu.VMEM((Q, 4), jnp.float32),
        pltpu.VMEM((16,), jnp.int32),
    ],
)
def _postprocess_sc(prob_hbm, boxes_hbm, ts_hbm,
                    oscore_hbm, olab_hbm, obox_hbm, *scratch):
  _body(prob_hbm, boxes_hbm, ts_hbm, oscore_hbm, olab_hbm, obox_hbm, *scratch)


def kernel(pred_logits, pred_boxes, target_sizes):
  prob = jax.nn.sigmoid(pred_logits).reshape(B, N)
  prob = jnp.pad(prob, ((0, 0), (0, NPAD - N)), constant_values=-1.0)
  pbits = lax.bitcast_convert_type(prob, jnp.int32)
  ts = jnp.pad(target_sizes, ((0, 0), (0, 14)))
  skey, labels, boxes = _postprocess_sc(pbits, pred_boxes, ts)
  scores = lax.bitcast_convert_type(skey[:, :K], jnp.float32)
  return (scores, labels[:, :K],
          boxes.reshape(B, OUTP, 4)[:, :K, :])
